# Initial kernel scaffold; baseline (speedup 1.0000x reference)
import jax
import jax.numpy as jnp
from jax.experimental import pallas as pl


def kernel(x, pos, edge_attr, W1, a1_src, a1_dst, b1, W2, a2_src, a2_dst, b2, Wp1, ap1_src, ap1_dst, bp1, Wp2, ap2_src, ap2_dst, bp2, M1, bm1, M2, bm2, edge_index, batch):
    raise NotImplementedError("write your pallas kernel here")


# speedup vs baseline: 28.0916x; 28.0916x over previous
import functools

import jax
import jax.numpy as jnp
import numpy as np
from jax import lax
from jax.experimental import pallas as pl
from jax.experimental.pallas import tpu as pltpu
from jax.experimental.pallas import tpu_sc as plsc

N_NODES = 10000
NP = 10240
N_EDGES = 320000
EP = 327680
NTILES = 32
EPT = EP // NTILES
EB = 512
NB = EPT // EB
NGRAPH = 16
ROWS_PER_TILE = NP // 16


def _tc_pre_body(v_ref, w_ref, ams_ref, amd_ref, h_ref, as_ref, ad_ref, m_ref):
    h = jnp.dot(v_ref[...], w_ref[...], preferred_element_type=jnp.float32)
    h_ref[...] = h
    As = jnp.dot(h, ams_ref[...], preferred_element_type=jnp.float32)
    Ad = jnp.dot(h, amd_ref[...], preferred_element_type=jnp.float32)
    as_ref[...] = As
    ad_ref[...] = Ad
    m = jnp.maximum(jnp.max(As, axis=0) + jnp.max(Ad, axis=0), 0.0)
    m_ref[...] = jnp.broadcast_to(m[None, :], (8, 16))


def _tc_pre(v, W, AmS, AmD):
    Fp = W.shape[1]
    return pl.pallas_call(
        _tc_pre_body,
        out_shape=(
            jax.ShapeDtypeStruct((NP, Fp), jnp.float32),
            jax.ShapeDtypeStruct((NP, 16), jnp.float32),
            jax.ShapeDtypeStruct((NP, 16), jnp.float32),
            jax.ShapeDtypeStruct((8, 16), jnp.float32),
        ),
    )(v, W, AmS, AmD)


@functools.cache
def _make_sc_edge(Fp, C):
    KF = Fp // 16
    mesh = plsc.VectorSubcoreMesh(core_axis_name="c", subcore_axis_name="s")

    @functools.partial(
        pl.kernel,
        out_type=(
            jax.ShapeDtypeStruct((2, NP, Fp), jnp.float32),
            jax.ShapeDtypeStruct((2, NP, 16), jnp.float32),
        ),
        mesh=mesh,
        compiler_params=pltpu.CompilerParams(
            use_tc_tiling_on_sc=False, needs_layout_passes=False),
        scratch_types=[
            pltpu.VMEM((EB,), jnp.int32),
            pltpu.VMEM((EB,), jnp.int32),
            pltpu.VMEM((EB, 16), jnp.float32),
            pltpu.VMEM((EB, 16), jnp.float32),
            pltpu.VMEM((EB, Fp), jnp.float32),
            pltpu.VMEM((EB * 16,), jnp.float32),
            pltpu.VMEM((EB, 16), jnp.float32),
            pltpu.VMEM((16,), jnp.float32),
            pltpu.VMEM_SHARED((NP, Fp), jnp.float32),
            pltpu.VMEM_SHARED((NP, 16), jnp.float32),
        ],
    )
    def sc_edge(h_hbm, as_hbm, ad_hbm, m_hbm, srcp_hbm, dstp_hbm,
                num_hbm, den_hbm,
                sidx, didx, asr, adr, rows, wbuf, wrows, mv, num_sh, den_sh):
        c = lax.axis_index("c")
        s = lax.axis_index("s")
        t = c * 16 + s
        zero16 = jnp.zeros((16,), jnp.float32)

        def _zb(b, carry):
            for k in range(KF):
                rows[b, pl.ds(k * 16, 16)] = zero16
            asr[b, :] = zero16
            return carry

        lax.fori_loop(0, EB, _zb, 0)
        r0 = s * ROWS_PER_TILE
        rem = ROWS_PER_TILE - EB
        pltpu.sync_copy(rows, num_sh.at[pl.ds(r0, EB)])
        pltpu.sync_copy(rows.at[pl.ds(0, rem)], num_sh.at[pl.ds(r0 + EB, rem)])
        pltpu.sync_copy(asr, den_sh.at[pl.ds(r0, EB)])
        pltpu.sync_copy(asr.at[pl.ds(0, rem)], den_sh.at[pl.ds(r0 + EB, rem)])
        pltpu.sync_copy(m_hbm.at[0], mv)
        plsc.subcore_barrier()

        lanes = lax.iota(jnp.int32, 16)

        def _batch(ib, carry):
            e0 = ib * EB
            pltpu.sync_copy(srcp_hbm.at[t, pl.ds(e0, EB)], sidx)
            pltpu.sync_copy(dstp_hbm.at[t, pl.ds(e0, EB)], didx)
            pltpu.sync_copy(as_hbm.at[sidx], asr)
            pltpu.sync_copy(ad_hbm.at[didx], adr)
            pltpu.sync_copy(h_hbm.at[sidx], rows)
            mvec = mv[...]

            def _edge(b, carry2):
                a = asr[b, :] + adr[b, :]
                al = jnp.maximum(a, 0.2 * a)
                w = jnp.exp(al - mvec)
                wbuf[pl.ds(b * 16, 16)] = w
                wrows[b, :] = w
                elanes = lax.iota(jnp.int32, 16)
                for k in range(KF):
                    pat = b * 16 + lax.div(elanes + k * 16, C)
                    wv = plsc.load_gather(wbuf, [pat])
                    rows[b, pl.ds(k * 16, 16)] = rows[b, pl.ds(k * 16, 16)] * wv
                return carry2

            lax.fori_loop(0, EB, _edge, 0)
            pltpu.sync_copy(wrows, den_sh.at[didx], add=True)
            pltpu.sync_copy(rows, num_sh.at[didx], add=True)
            return carry

        lax.fori_loop(0, NB, _batch, 0)
        plsc.subcore_barrier()
        pltpu.sync_copy(num_sh.at[pl.ds(r0, ROWS_PER_TILE)],
                        num_hbm.at[c, pl.ds(r0, ROWS_PER_TILE)])
        pltpu.sync_copy(den_sh.at[pl.ds(r0, ROWS_PER_TILE)],
                        den_hbm.at[c, pl.ds(r0, ROWS_PER_TILE)])

    return sc_edge


def _tc_post_body(num_ref, den_ref, emat_ref, b_ref, out_ref, *, relu):
    ns = num_ref[0] + num_ref[1]
    dsum = den_ref[0] + den_ref[1]
    de = jnp.dot(dsum, emat_ref[...], preferred_element_type=jnp.float32)
    o = ns / (de + 1e-30) + b_ref[...]
    if relu:
        o = jnp.maximum(o, 0.0)
    out_ref[...] = o


def _tc_post(num, den, Emat, brow, relu):
    Fp = num.shape[2]
    return pl.pallas_call(
        functools.partial(_tc_post_body, relu=relu),
        out_shape=jax.ShapeDtypeStruct((NP, Fp), jnp.float32),
    )(num, den, Emat, brow)


def _tc_post2_body(numa_ref, numb_ref, den_ref, emat_ref, b_ref, out_ref):
    ns = jnp.concatenate([numa_ref[0] + numa_ref[1],
                          numb_ref[0] + numb_ref[1]], axis=1)
    dsum = den_ref[0] + den_ref[1]
    de = jnp.dot(dsum, emat_ref[...], preferred_element_type=jnp.float32)
    out_ref[...] = ns / (de + 1e-30) + b_ref[...]


def _tc_post2(numa, numb, den, Emat, brow):
    return pl.pallas_call(
        _tc_post2_body,
        out_shape=jax.ShapeDtypeStruct((NP, 128), jnp.float32),
    )(numa, numb, den, Emat, brow)


def _tc_tail_body(x_ref, p_ref, batch_ref, m1x_ref, m1p_ref, bm1_ref,
                  m2_ref, bm2_ref, out_ref):
    bvec = batch_ref[...]
    gids = lax.broadcasted_iota(jnp.int32, (NGRAPH, 1), 0)
    onehot = (bvec == gids).astype(jnp.float32)
    cnt = jnp.maximum(jnp.sum(onehot, axis=1, keepdims=True), 1.0)
    xg = jnp.dot(onehot, x_ref[...], preferred_element_type=jnp.float32) / cnt
    pg = jnp.dot(onehot, p_ref[...], preferred_element_type=jnp.float32) / cnt
    h1 = (jnp.dot(xg, m1x_ref[...], preferred_element_type=jnp.float32)
          + jnp.dot(pg, m1p_ref[...], preferred_element_type=jnp.float32)
          + bm1_ref[...])
    h1 = jnp.maximum(h1, 0.0)
    out_ref[...] = (jnp.dot(h1, m2_ref[...], preferred_element_type=jnp.float32)
                    + bm2_ref[...])


def _tc_tail(xp, pp, batch_p, M1x, M1p, bm1r, M2, bm2r):
    return pl.pallas_call(
        _tc_tail_body,
        out_shape=jax.ShapeDtypeStruct((NGRAPH, 10), jnp.float32),
    )(xp, pp, batch_p, M1x, M1p, bm1r, M2, bm2r)


def _attn_mats(a, Fp):
    H, C = a.shape
    Am = jnp.zeros((Fp, 16), jnp.float32)
    rows = jnp.arange(H * C)
    cols = jnp.repeat(jnp.arange(H), C)
    return Am.at[rows, cols].set(a.reshape(-1))


def _expand_mat(H, C, Fp):
    Em = jnp.zeros((16, Fp), jnp.float32)
    rows = jnp.repeat(jnp.arange(H), C)
    cols = jnp.arange(H * C)
    return Em.at[rows, cols].set(1.0)


def _pad_bias(b, Fp):
    return jnp.zeros((1, Fp), jnp.float32).at[0, : b.shape[0]].set(b)


def kernel(x, pos, edge_attr, W1, a1_src, a1_dst, b1, W2, a2_src, a2_dst, b2,
           Wp1, ap1_src, ap1_dst, bp1, Wp2, ap2_src, ap2_dst, bp2,
           M1, bm1, M2, bm2, edge_index, batch):
    del edge_attr
    f32 = jnp.float32

    src = edge_index[0]
    dst = edge_index[1]
    pad_e = jnp.full((EP - N_EDGES,), N_NODES, jnp.int32)
    srcp = jnp.concatenate([src, pad_e]).reshape(NTILES, EPT)
    dstp = jnp.concatenate([dst, pad_e]).reshape(NTILES, EPT)

    xp = jnp.zeros((NP, 128), f32).at[:N_NODES].set(x)
    pp = jnp.zeros((NP, 16), f32).at[:N_NODES, :3].set(pos)
    batch_p = jnp.full((1, NP), NGRAPH, jnp.int32).at[0, :N_NODES].set(batch)

    W1p = W1
    W2p = W2
    Wp1p = jnp.zeros((16, 64), f32).at[:3].set(Wp1)
    Wp2p = jnp.zeros((64, 16), f32).at[:, :3].set(Wp2)

    convs = {
        "x1": dict(W=W1p, AmS=_attn_mats(a1_src, 64), AmD=_attn_mats(a1_dst, 64),
                   Em=_expand_mat(8, 8, 64), b=_pad_bias(b1, 64), Fp=64, C=8,
                   relu=True),
        "x2": dict(W=W2p, AmS=_attn_mats(a2_src, 128), AmD=_attn_mats(a2_dst, 128),
                   Em=_expand_mat(1, 128, 128), b=_pad_bias(b2, 128)),
        "p1": dict(W=Wp1p, AmS=_attn_mats(ap1_src, 64), AmD=_attn_mats(ap1_dst, 64),
                   Em=_expand_mat(8, 8, 64), b=_pad_bias(bp1, 64), Fp=64, C=8,
                   relu=True),
        "p2": dict(W=Wp2p, AmS=_attn_mats(ap2_src, 16), AmD=_attn_mats(ap2_dst, 16),
                   Em=_expand_mat(1, 3, 16), b=_pad_bias(bp2, 16), Fp=16, C=3,
                   relu=False),
    }

    def conv(v, cfg):
        h, As, Ad, m = _tc_pre(v, cfg["W"], cfg["AmS"], cfg["AmD"])
        num, den = _make_sc_edge(cfg["Fp"], cfg["C"])(h, As, Ad, m, srcp, dstp)
        return _tc_post(num, den, cfg["Em"], cfg["b"], cfg["relu"])

    def conv_x2(v, cfg):
        h, As, Ad, m = _tc_pre(v, cfg["W"], cfg["AmS"], cfg["AmD"])
        ha = lax.slice_in_dim(h, 0, 64, axis=1)
        hb = lax.slice_in_dim(h, 64, 128, axis=1)
        sc = _make_sc_edge(64, 64)
        numa, den = sc(ha, As, Ad, m, srcp, dstp)
        numb, _ = sc(hb, As, Ad, m, srcp, dstp)
        return _tc_post2(numa, numb, den, cfg["Em"], cfg["b"])

    for _ in range(5):
        pp = conv(pp, convs["p1"])
        pp = conv(pp, convs["p2"])
        xp = conv(xp, convs["x1"])
        xp = conv_x2(xp, convs["x2"])

    M1x = M1[:128]
    M1p = jnp.zeros((16, 65), f32).at[:3].set(M1[128:131])
    bm1r = bm1.reshape(1, 65)
    bm2r = bm2.reshape(1, 10)
    return _tc_tail(xp, pp, batch_p, M1x, M1p, bm1r, M2, bm2r)


# speedup vs baseline: 44.8503x; 1.5966x over previous
import functools

import jax
import jax.numpy as jnp
import numpy as np
from jax import lax
from jax.experimental import pallas as pl
from jax.experimental.pallas import tpu as pltpu
from jax.experimental.pallas import tpu_sc as plsc

N_NODES = 10000
NP = 10240
N_EDGES = 320000
EP = 327680
NTILES = 32
EPT = EP // NTILES
EB = 256
NB = EPT // EB
NGRAPH = 16
ROWS_PER_TILE = NP // 16


def _tc_pre_body(v_ref, w_ref, ams_ref, amd_ref, hp_ref, ad_ref, m_ref):
    h = jnp.dot(v_ref[...], w_ref[...], preferred_element_type=jnp.float32)
    As = jnp.dot(h, ams_ref[...], preferred_element_type=jnp.float32)
    Ad = jnp.dot(h, amd_ref[...], preferred_element_type=jnp.float32)
    hp_ref[...] = jnp.concatenate([h, As], axis=1)
    ad_ref[...] = Ad
    m = jnp.maximum(jnp.max(As, axis=0) + jnp.max(Ad, axis=0), 0.0)
    m_ref[...] = jnp.broadcast_to(m[None, :], (8, 16))


def _tc_pre(v, W, AmS, AmD):
    Fp = W.shape[1]
    return pl.pallas_call(
        _tc_pre_body,
        out_shape=(
            jax.ShapeDtypeStruct((NP, Fp + 16), jnp.float32),
            jax.ShapeDtypeStruct((NP, 16), jnp.float32),
            jax.ShapeDtypeStruct((8, 16), jnp.float32),
        ),
    )(v, W, AmS, AmD)


def _tc_pre_x2_body(v_ref, w_ref, ams_ref, amd_ref, hpa_ref, hpb_ref, ad_ref,
                    m_ref):
    h = jnp.dot(v_ref[...], w_ref[...], preferred_element_type=jnp.float32)
    As = jnp.dot(h, ams_ref[...], preferred_element_type=jnp.float32)
    Ad = jnp.dot(h, amd_ref[...], preferred_element_type=jnp.float32)
    hpa_ref[...] = jnp.concatenate([h[:, :64], As], axis=1)
    hpb_ref[...] = jnp.concatenate([h[:, 64:], As], axis=1)
    ad_ref[...] = Ad
    m = jnp.maximum(jnp.max(As, axis=0) + jnp.max(Ad, axis=0), 0.0)
    m_ref[...] = jnp.broadcast_to(m[None, :], (8, 16))


def _tc_pre_x2(v, W, AmS, AmD):
    return pl.pallas_call(
        _tc_pre_x2_body,
        out_shape=(
            jax.ShapeDtypeStruct((NP, 80), jnp.float32),
            jax.ShapeDtypeStruct((NP, 80), jnp.float32),
            jax.ShapeDtypeStruct((NP, 16), jnp.float32),
            jax.ShapeDtypeStruct((8, 16), jnp.float32),
        ),
    )(v, W, AmS, AmD)


@functools.cache
def _make_sc_edge(Fp, C):
    KF = Fp // 16
    FT = Fp + 16
    mesh = plsc.VectorSubcoreMesh(core_axis_name="c", subcore_axis_name="s")

    @functools.partial(
        pl.kernel,
        out_type=jax.ShapeDtypeStruct((2, NP, FT), jnp.float32),
        mesh=mesh,
        compiler_params=pltpu.CompilerParams(
            use_tc_tiling_on_sc=False, needs_layout_passes=False),
        scratch_types=[
            pltpu.VMEM((EB,), jnp.int32),
            pltpu.VMEM((EB,), jnp.int32),
            pltpu.VMEM((EB,), jnp.int32),
            pltpu.VMEM((EB,), jnp.int32),
            pltpu.VMEM((EB, FT), jnp.float32),
            pltpu.VMEM((EB, FT), jnp.float32),
            pltpu.VMEM((EB, 16), jnp.float32),
            pltpu.VMEM((EB, 16), jnp.float32),
            pltpu.VMEM((EB * 16,), jnp.float32),
            pltpu.VMEM((16,), jnp.float32),
            pltpu.VMEM_SHARED((NP, FT), jnp.float32),
            pltpu.SemaphoreType.DMA,
            pltpu.SemaphoreType.DMA,
            pltpu.SemaphoreType.DMA,
            pltpu.SemaphoreType.DMA,
        ],
    )
    def sc_edge(hp_hbm, ad_hbm, m_hbm, srcp_hbm, dstp_hbm, nd_hbm,
                sidx0, sidx1, didx0, didx1, rows0, rows1, adr0, adr1,
                wflat, mv, nd_sh, gsem0, gsem1, ssem0, ssem1):
        c = lax.axis_index("c")
        s = lax.axis_index("s")
        t = c * 16 + s
        sidx = (sidx0, sidx1)
        didx = (didx0, didx1)
        rows = (rows0, rows1)
        adr = (adr0, adr1)
        gsems = (gsem0, gsem1)
        ssems = (ssem0, ssem1)
        zero16 = jnp.zeros((16,), jnp.float32)

        def _zb(b, carry):
            for k in range(KF + 1):
                rows0[b, pl.ds(k * 16, 16)] = zero16
            return carry

        lax.fori_loop(0, EB, _zb, 0)
        r0 = s * ROWS_PER_TILE
        off = 0
        while off < ROWS_PER_TILE:
            sz = min(EB, ROWS_PER_TILE - off)
            pltpu.sync_copy(rows0.at[pl.ds(0, sz)],
                            nd_sh.at[pl.ds(r0 + off, sz)])
            off += sz
        pltpu.sync_copy(m_hbm.at[0], mv)
        plsc.subcore_barrier()

        def _start_gathers(ib, p):
            e0 = ib * EB
            pltpu.sync_copy(srcp_hbm.at[t, pl.ds(e0, EB)], sidx[p])
            pltpu.sync_copy(dstp_hbm.at[t, pl.ds(e0, EB)], didx[p])
            pltpu.make_async_copy(hp_hbm.at[sidx[p]], rows[p],
                                  gsems[p]).start()
            pltpu.make_async_copy(ad_hbm.at[didx[p]], adr[p],
                                  gsems[p]).start()

        def _wait_gathers(p):
            pltpu.make_async_copy(hp_hbm.at[sidx[p]], rows[p],
                                  gsems[p]).wait()
            pltpu.make_async_copy(ad_hbm.at[didx[p]], adr[p],
                                  gsems[p]).wait()

        def _start_scatter(p):
            pltpu.make_async_copy(rows[p], nd_sh.at[didx[p]],
                                  ssems[p]).start(add=True)

        def _wait_scatter(p):
            pltpu.make_async_copy(rows[p], nd_sh.at[didx[p]],
                                  ssems[p]).wait()

        def _proc(ib, p):
            q = 1 - p
            _wait_gathers(p)

            @pl.when(ib >= 1)
            def _():
                _wait_scatter(q)

            @pl.when(ib + 1 < NB)
            def _():
                _start_gathers(ib + 1, q)

            mvec = mv[...]
            rp = rows[p]
            ap = adr[p]

            def _edge(b, carry2):
                a = rp[b, pl.ds(Fp, 16)] + ap[b, :]
                al = jnp.maximum(a, 0.2 * a)
                w = jnp.exp(al - mvec)
                rp[b, pl.ds(Fp, 16)] = w
                wflat[pl.ds(b * 16, 16)] = w
                elanes = lax.iota(jnp.int32, 16)
                for k in range(KF):
                    pat = b * 16 + lax.div(elanes + k * 16, C)
                    wv = plsc.load_gather(wflat, [pat])
                    rp[b, pl.ds(k * 16, 16)] = rp[b, pl.ds(k * 16, 16)] * wv
                return carry2

            lax.fori_loop(0, EB, _edge, 0, unroll=8)
            _start_scatter(p)

        _start_gathers(0, 0)

        def _pair(j, carry):
            _proc(j * 2, 0)
            _proc(j * 2 + 1, 1)
            return carry

        lax.fori_loop(0, NB // 2, _pair, 0)
        _wait_scatter((NB - 1) % 2)
        plsc.subcore_barrier()
        pltpu.sync_copy(nd_sh.at[pl.ds(r0, ROWS_PER_TILE)],
                        nd_hbm.at[c, pl.ds(r0, ROWS_PER_TILE)])

    return sc_edge


def _tc_post_body(nd_ref, emat_ref, b_ref, out_ref, *, Fp, relu):
    ns = nd_ref[0] + nd_ref[1]
    num = ns[:, :Fp]
    den = ns[:, Fp:]
    de = jnp.dot(den, emat_ref[...], preferred_element_type=jnp.float32)
    o = num / (de + 1e-30) + b_ref[...]
    if relu:
        o = jnp.maximum(o, 0.0)
    out_ref[...] = o


def _tc_post(nd, Emat, brow, relu):
    Fp = nd.shape[2] - 16
    return pl.pallas_call(
        functools.partial(_tc_post_body, Fp=Fp, relu=relu),
        out_shape=jax.ShapeDtypeStruct((NP, Fp), jnp.float32),
    )(nd, Emat, brow)


def _tc_post2_body(nda_ref, ndb_ref, emat_ref, b_ref, out_ref):
    sa = nda_ref[0] + nda_ref[1]
    sb = ndb_ref[0] + ndb_ref[1]
    num = jnp.concatenate([sa[:, :64], sb[:, :64]], axis=1)
    den = sa[:, 64:]
    de = jnp.dot(den, emat_ref[...], preferred_element_type=jnp.float32)
    out_ref[...] = num / (de + 1e-30) + b_ref[...]


def _tc_post2(nda, ndb, Emat, brow):
    return pl.pallas_call(
        _tc_post2_body,
        out_shape=jax.ShapeDtypeStruct((NP, 128), jnp.float32),
    )(nda, ndb, Emat, brow)


def _tc_tail_body(x_ref, p_ref, batch_ref, m1x_ref, m1p_ref, bm1_ref,
                  m2_ref, bm2_ref, out_ref):
    bvec = batch_ref[...]
    gids = lax.broadcasted_iota(jnp.int32, (NGRAPH, 1), 0)
    onehot = (bvec == gids).astype(jnp.float32)
    cnt = jnp.maximum(jnp.sum(onehot, axis=1, keepdims=True), 1.0)
    xg = jnp.dot(onehot, x_ref[...], preferred_element_type=jnp.float32) / cnt
    pg = jnp.dot(onehot, p_ref[...], preferred_element_type=jnp.float32) / cnt
    h1 = (jnp.dot(xg, m1x_ref[...], preferred_element_type=jnp.float32)
          + jnp.dot(pg, m1p_ref[...], preferred_element_type=jnp.float32)
          + bm1_ref[...])
    h1 = jnp.maximum(h1, 0.0)
    out_ref[...] = (jnp.dot(h1, m2_ref[...], preferred_element_type=jnp.float32)
                    + bm2_ref[...])


def _tc_tail(xp, pp, batch_p, M1x, M1p, bm1r, M2, bm2r):
    return pl.pallas_call(
        _tc_tail_body,
        out_shape=jax.ShapeDtypeStruct((NGRAPH, 10), jnp.float32),
    )(xp, pp, batch_p, M1x, M1p, bm1r, M2, bm2r)


def _attn_mats(a, Fp):
    H, C = a.shape
    Am = jnp.zeros((Fp, 16), jnp.float32)
    rows = jnp.arange(H * C)
    cols = jnp.repeat(jnp.arange(H), C)
    return Am.at[rows, cols].set(a.reshape(-1))


def _expand_mat(H, C, Fp):
    Em = jnp.zeros((16, Fp), jnp.float32)
    rows = jnp.repeat(jnp.arange(H), C)
    cols = jnp.arange(H * C)
    return Em.at[rows, cols].set(1.0)


def _pad_bias(b, Fp):
    return jnp.zeros((1, Fp), jnp.float32).at[0, : b.shape[0]].set(b)


def kernel(x, pos, edge_attr, W1, a1_src, a1_dst, b1, W2, a2_src, a2_dst, b2,
           Wp1, ap1_src, ap1_dst, bp1, Wp2, ap2_src, ap2_dst, bp2,
           M1, bm1, M2, bm2, edge_index, batch):
    del edge_attr
    f32 = jnp.float32

    src = edge_index[0]
    dst = edge_index[1]
    pad_e = jnp.full((EP - N_EDGES,), N_NODES, jnp.int32)
    srcp = jnp.concatenate([src, pad_e]).reshape(NTILES, EPT)
    dstp = jnp.concatenate([dst, pad_e]).reshape(NTILES, EPT)

    xp = jnp.zeros((NP, 128), f32).at[:N_NODES].set(x)
    pp = jnp.zeros((NP, 16), f32).at[:N_NODES, :3].set(pos)
    batch_p = jnp.full((1, NP), NGRAPH, jnp.int32).at[0, :N_NODES].set(batch)

    Wp1p = jnp.zeros((16, 64), f32).at[:3].set(Wp1)
    Wp2p = jnp.zeros((64, 16), f32).at[:, :3].set(Wp2)

    convs = {
        "x1": dict(W=W1, AmS=_attn_mats(a1_src, 64), AmD=_attn_mats(a1_dst, 64),
                   Em=_expand_mat(8, 8, 64), b=_pad_bias(b1, 64), Fp=64, C=8,
                   relu=True),
        "x2": dict(W=W2, AmS=_attn_mats(a2_src, 128), AmD=_attn_mats(a2_dst, 128),
                   Em=_expand_mat(1, 128, 128), b=_pad_bias(b2, 128)),
        "p1": dict(W=Wp1p, AmS=_attn_mats(ap1_src, 64), AmD=_attn_mats(ap1_dst, 64),
                   Em=_expand_mat(8, 8, 64), b=_pad_bias(bp1, 64), Fp=64, C=8,
                   relu=True),
        "p2": dict(W=Wp2p, AmS=_attn_mats(ap2_src, 16), AmD=_attn_mats(ap2_dst, 16),
                   Em=_expand_mat(1, 3, 16), b=_pad_bias(bp2, 16), Fp=16, C=3,
                   relu=False),
    }

    def conv(v, cfg):
        hp, Ad, m = _tc_pre(v, cfg["W"], cfg["AmS"], cfg["AmD"])
        nd = _make_sc_edge(cfg["Fp"], cfg["C"])(hp, Ad, m, srcp, dstp)
        return _tc_post(nd, cfg["Em"], cfg["b"], cfg["relu"])

    def conv_x2(v, cfg):
        hpa, hpb, Ad, m = _tc_pre_x2(v, cfg["W"], cfg["AmS"], cfg["AmD"])
        sc = _make_sc_edge(64, 64)
        nda = sc(hpa, Ad, m, srcp, dstp)
        ndb = sc(hpb, Ad, m, srcp, dstp)
        return _tc_post2(nda, ndb, cfg["Em"], cfg["b"])

    for _ in range(5):
        pp = conv(pp, convs["p1"])
        pp = conv(pp, convs["p2"])
        xp = conv(xp, convs["x1"])
        xp = conv_x2(xp, convs["x2"])

    M1x = M1[:128]
    M1p = jnp.zeros((16, 65), f32).at[:3].set(M1[128:131])
    bm1r = bm1.reshape(1, 65)
    bm2r = bm2.reshape(1, 10)
    return _tc_tail(xp, pp, batch_p, M1x, M1p, bm1r, M2, bm2r)


# speedup vs baseline: 49.1823x; 1.0966x over previous
import functools

import jax
import jax.numpy as jnp
import numpy as np
from jax import lax
from jax.experimental import pallas as pl
from jax.experimental.pallas import tpu as pltpu
from jax.experimental.pallas import tpu_sc as plsc

N_NODES = 10000
NP = 10240
N_EDGES = 320000
EP = 327680
NTILES = 32
EPT = EP // NTILES
EB = 256
NB = EPT // EB
NGRAPH = 16
ROWS_PER_TILE = NP // 16


def _tc_pre_body(v_ref, w_ref, ams_ref, amd_ref, hp_ref, ad_ref, m_ref):
    h = jnp.dot(v_ref[...], w_ref[...], preferred_element_type=jnp.float32)
    As = jnp.dot(h, ams_ref[...], preferred_element_type=jnp.float32)
    Ad = jnp.dot(h, amd_ref[...], preferred_element_type=jnp.float32)
    hp_ref[...] = jnp.concatenate([h, As], axis=1)
    ad_ref[...] = Ad
    m = jnp.maximum(jnp.max(As, axis=0) + jnp.max(Ad, axis=0), 0.0)
    m_ref[...] = jnp.broadcast_to(m[None, :], (8, 16))


def _tc_pre(v, W, AmS, AmD):
    Fp = W.shape[1]
    return pl.pallas_call(
        _tc_pre_body,
        out_shape=(
            jax.ShapeDtypeStruct((NP, Fp + 16), jnp.float32),
            jax.ShapeDtypeStruct((NP, 16), jnp.float32),
            jax.ShapeDtypeStruct((8, 16), jnp.float32),
        ),
    )(v, W, AmS, AmD)


def _tc_pre_x2_body(v_ref, w_ref, ams_ref, amd_ref, hpa_ref, hpb_ref, ad_ref,
                    m_ref):
    h = jnp.dot(v_ref[...], w_ref[...], preferred_element_type=jnp.float32)
    As = jnp.dot(h, ams_ref[...], preferred_element_type=jnp.float32)
    Ad = jnp.dot(h, amd_ref[...], preferred_element_type=jnp.float32)
    hpa_ref[...] = jnp.concatenate([h[:, :64], As], axis=1)
    hpb_ref[...] = jnp.concatenate([h[:, 64:], As], axis=1)
    ad_ref[...] = Ad
    m = jnp.maximum(jnp.max(As, axis=0) + jnp.max(Ad, axis=0), 0.0)
    m_ref[...] = jnp.broadcast_to(m[None, :], (8, 16))


def _tc_pre_x2(v, W, AmS, AmD):
    return pl.pallas_call(
        _tc_pre_x2_body,
        out_shape=(
            jax.ShapeDtypeStruct((NP, 80), jnp.float32),
            jax.ShapeDtypeStruct((NP, 80), jnp.float32),
            jax.ShapeDtypeStruct((NP, 16), jnp.float32),
            jax.ShapeDtypeStruct((8, 16), jnp.float32),
        ),
    )(v, W, AmS, AmD)


@functools.cache
def _make_sc_edge(Fp, C):
    KF = Fp // 16
    FT = Fp + 16
    mesh = plsc.VectorSubcoreMesh(core_axis_name="c", subcore_axis_name="s")

    @functools.partial(
        pl.kernel,
        out_type=jax.ShapeDtypeStruct((2, NP, FT), jnp.float32),
        mesh=mesh,
        compiler_params=pltpu.CompilerParams(
            use_tc_tiling_on_sc=False, needs_layout_passes=False),
        scratch_types=[
            pltpu.VMEM((EB,), jnp.int32),
            pltpu.VMEM((EB,), jnp.int32),
            pltpu.VMEM((EB,), jnp.int32),
            pltpu.VMEM((EB,), jnp.int32),
            pltpu.VMEM((EB, FT), jnp.float32),
            pltpu.VMEM((EB, FT), jnp.float32),
            pltpu.VMEM((EB, 16), jnp.float32),
            pltpu.VMEM((EB, 16), jnp.float32),
            pltpu.VMEM((16,), jnp.float32),
            pltpu.VMEM_SHARED((NP, FT), jnp.float32),
            pltpu.SemaphoreType.DMA,
            pltpu.SemaphoreType.DMA,
            pltpu.SemaphoreType.DMA,
            pltpu.SemaphoreType.DMA,
        ],
    )
    def sc_edge(hp_hbm, ad_hbm, m_hbm, srcp_hbm, dstp_hbm, nd_hbm,
                sidx0, sidx1, didx0, didx1, rows0, rows1, adr0, adr1,
                mv, nd_sh, gsem0, gsem1, ssem0, ssem1):
        c = lax.axis_index("c")
        s = lax.axis_index("s")
        t = c * 16 + s
        sidx = (sidx0, sidx1)
        didx = (didx0, didx1)
        rows = (rows0, rows1)
        adr = (adr0, adr1)
        gsems = (gsem0, gsem1)
        ssems = (ssem0, ssem1)
        zero16 = jnp.zeros((16,), jnp.float32)

        def _zb(b, carry):
            for k in range(KF + 1):
                rows0[b, pl.ds(k * 16, 16)] = zero16
            return carry

        lax.fori_loop(0, EB, _zb, 0)
        r0 = s * ROWS_PER_TILE
        off = 0
        while off < ROWS_PER_TILE:
            sz = min(EB, ROWS_PER_TILE - off)
            pltpu.sync_copy(rows0.at[pl.ds(0, sz)],
                            nd_sh.at[pl.ds(r0 + off, sz)])
            off += sz
        pltpu.sync_copy(m_hbm.at[0], mv)
        plsc.subcore_barrier()

        def _start_gathers(ib, p):
            e0 = ib * EB
            pltpu.sync_copy(srcp_hbm.at[t, pl.ds(e0, EB)], sidx[p])
            pltpu.sync_copy(dstp_hbm.at[t, pl.ds(e0, EB)], didx[p])
            pltpu.make_async_copy(hp_hbm.at[sidx[p]], rows[p],
                                  gsems[p]).start()
            pltpu.make_async_copy(ad_hbm.at[didx[p]], adr[p],
                                  gsems[p]).start()

        def _wait_gathers(p):
            pltpu.make_async_copy(hp_hbm.at[sidx[p]], rows[p],
                                  gsems[p]).wait()
            pltpu.make_async_copy(ad_hbm.at[didx[p]], adr[p],
                                  gsems[p]).wait()

        def _start_scatter(p):
            pltpu.make_async_copy(rows[p], nd_sh.at[didx[p]],
                                  ssems[p]).start(add=True)

        def _wait_scatter(p):
            pltpu.make_async_copy(rows[p], nd_sh.at[didx[p]],
                                  ssems[p]).wait()

        def _proc(ib, p):
            q = 1 - p
            _wait_gathers(p)

            @pl.when(ib >= 1)
            def _():
                _wait_scatter(q)

            @pl.when(ib + 1 < NB)
            def _():
                _start_gathers(ib + 1, q)

            mvec = mv[...]
            rp = rows[p]
            ap = adr[p]

            def _edge(b, carry2):
                a = rp[b, pl.ds(Fp, 16)] + ap[b, :]
                al = jnp.maximum(a, 0.2 * a)
                w = jnp.exp(al - mvec)
                rp[b, pl.ds(Fp, 16)] = w
                elanes = lax.iota(jnp.int32, 16)
                for k in range(KF):
                    pat = lax.div(elanes + k * 16, C)
                    wv = w.at[pat].get(mode="promise_in_bounds")
                    rp[b, pl.ds(k * 16, 16)] = rp[b, pl.ds(k * 16, 16)] * wv
                return carry2

            lax.fori_loop(0, EB, _edge, 0, unroll=8)
            _start_scatter(p)

        _start_gathers(0, 0)

        def _pair(j, carry):
            _proc(j * 2, 0)
            _proc(j * 2 + 1, 1)
            return carry

        lax.fori_loop(0, NB // 2, _pair, 0)
        _wait_scatter((NB - 1) % 2)
        plsc.subcore_barrier()
        pltpu.sync_copy(nd_sh.at[pl.ds(r0, ROWS_PER_TILE)],
                        nd_hbm.at[c, pl.ds(r0, ROWS_PER_TILE)])

    return sc_edge


def _tc_post_body(nd_ref, emat_ref, b_ref, out_ref, *, Fp, relu):
    ns = nd_ref[0] + nd_ref[1]
    num = ns[:, :Fp]
    den = ns[:, Fp:]
    de = jnp.dot(den, emat_ref[...], preferred_element_type=jnp.float32)
    o = num / (de + 1e-30) + b_ref[...]
    if relu:
        o = jnp.maximum(o, 0.0)
    out_ref[...] = o


def _tc_post(nd, Emat, brow, relu):
    Fp = nd.shape[2] - 16
    return pl.pallas_call(
        functools.partial(_tc_post_body, Fp=Fp, relu=relu),
        out_shape=jax.ShapeDtypeStruct((NP, Fp), jnp.float32),
    )(nd, Emat, brow)


def _tc_post2_body(nda_ref, ndb_ref, emat_ref, b_ref, out_ref):
    sa = nda_ref[0] + nda_ref[1]
    sb = ndb_ref[0] + ndb_ref[1]
    num = jnp.concatenate([sa[:, :64], sb[:, :64]], axis=1)
    den = sa[:, 64:]
    de = jnp.dot(den, emat_ref[...], preferred_element_type=jnp.float32)
    out_ref[...] = num / (de + 1e-30) + b_ref[...]


def _tc_post2(nda, ndb, Emat, brow):
    return pl.pallas_call(
        _tc_post2_body,
        out_shape=jax.ShapeDtypeStruct((NP, 128), jnp.float32),
    )(nda, ndb, Emat, brow)


def _tc_tail_body(x_ref, p_ref, batch_ref, m1x_ref, m1p_ref, bm1_ref,
                  m2_ref, bm2_ref, out_ref):
    bvec = batch_ref[...]
    gids = lax.broadcasted_iota(jnp.int32, (NGRAPH, 1), 0)
    onehot = (bvec == gids).astype(jnp.float32)
    cnt = jnp.maximum(jnp.sum(onehot, axis=1, keepdims=True), 1.0)
    xg = jnp.dot(onehot, x_ref[...], preferred_element_type=jnp.float32) / cnt
    pg = jnp.dot(onehot, p_ref[...], preferred_element_type=jnp.float32) / cnt
    h1 = (jnp.dot(xg, m1x_ref[...], preferred_element_type=jnp.float32)
          + jnp.dot(pg, m1p_ref[...], preferred_element_type=jnp.float32)
          + bm1_ref[...])
    h1 = jnp.maximum(h1, 0.0)
    out_ref[...] = (jnp.dot(h1, m2_ref[...], preferred_element_type=jnp.float32)
                    + bm2_ref[...])


def _tc_tail(xp, pp, batch_p, M1x, M1p, bm1r, M2, bm2r):
    return pl.pallas_call(
        _tc_tail_body,
        out_shape=jax.ShapeDtypeStruct((NGRAPH, 10), jnp.float32),
    )(xp, pp, batch_p, M1x, M1p, bm1r, M2, bm2r)


def _attn_mats(a, Fp):
    H, C = a.shape
    Am = jnp.zeros((Fp, 16), jnp.float32)
    rows = jnp.arange(H * C)
    cols = jnp.repeat(jnp.arange(H), C)
    return Am.at[rows, cols].set(a.reshape(-1))


def _expand_mat(H, C, Fp):
    Em = jnp.zeros((16, Fp), jnp.float32)
    rows = jnp.repeat(jnp.arange(H), C)
    cols = jnp.arange(H * C)
    return Em.at[rows, cols].set(1.0)


def _pad_bias(b, Fp):
    return jnp.zeros((1, Fp), jnp.float32).at[0, : b.shape[0]].set(b)


def kernel(x, pos, edge_attr, W1, a1_src, a1_dst, b1, W2, a2_src, a2_dst, b2,
           Wp1, ap1_src, ap1_dst, bp1, Wp2, ap2_src, ap2_dst, bp2,
           M1, bm1, M2, bm2, edge_index, batch):
    del edge_attr
    f32 = jnp.float32

    src = edge_index[0]
    dst = edge_index[1]
    pad_e = jnp.full((EP - N_EDGES,), N_NODES, jnp.int32)
    srcp = jnp.concatenate([src, pad_e]).reshape(NTILES, EPT)
    dstp = jnp.concatenate([dst, pad_e]).reshape(NTILES, EPT)

    xp = jnp.zeros((NP, 128), f32).at[:N_NODES].set(x)
    pp = jnp.zeros((NP, 16), f32).at[:N_NODES, :3].set(pos)
    batch_p = jnp.full((1, NP), NGRAPH, jnp.int32).at[0, :N_NODES].set(batch)

    Wp1p = jnp.zeros((16, 64), f32).at[:3].set(Wp1)
    Wp2p = jnp.zeros((64, 16), f32).at[:, :3].set(Wp2)

    convs = {
        "x1": dict(W=W1, AmS=_attn_mats(a1_src, 64), AmD=_attn_mats(a1_dst, 64),
                   Em=_expand_mat(8, 8, 64), b=_pad_bias(b1, 64), Fp=64, C=8,
                   relu=True),
        "x2": dict(W=W2, AmS=_attn_mats(a2_src, 128), AmD=_attn_mats(a2_dst, 128),
                   Em=_expand_mat(1, 128, 128), b=_pad_bias(b2, 128)),
        "p1": dict(W=Wp1p, AmS=_attn_mats(ap1_src, 64), AmD=_attn_mats(ap1_dst, 64),
                   Em=_expand_mat(8, 8, 64), b=_pad_bias(bp1, 64), Fp=64, C=8,
                   relu=True),
        "p2": dict(W=Wp2p, AmS=_attn_mats(ap2_src, 16), AmD=_attn_mats(ap2_dst, 16),
                   Em=_expand_mat(1, 3, 16), b=_pad_bias(bp2, 16), Fp=16, C=3,
                   relu=False),
    }

    def conv(v, cfg):
        hp, Ad, m = _tc_pre(v, cfg["W"], cfg["AmS"], cfg["AmD"])
        nd = _make_sc_edge(cfg["Fp"], cfg["C"])(hp, Ad, m, srcp, dstp)
        return _tc_post(nd, cfg["Em"], cfg["b"], cfg["relu"])

    def conv_x2(v, cfg):
        hpa, hpb, Ad, m = _tc_pre_x2(v, cfg["W"], cfg["AmS"], cfg["AmD"])
        sc = _make_sc_edge(64, 64)
        nda = sc(hpa, Ad, m, srcp, dstp)
        ndb = sc(hpb, Ad, m, srcp, dstp)
        return _tc_post2(nda, ndb, cfg["Em"], cfg["b"])

    for _ in range(5):
        pp = conv(pp, convs["p1"])
        pp = conv(pp, convs["p2"])
        xp = conv(xp, convs["x1"])
        xp = conv_x2(xp, convs["x2"])

    M1x = M1[:128]
    M1p = jnp.zeros((16, 65), f32).at[:3].set(M1[128:131])
    bm1r = bm1.reshape(1, 65)
    bm2r = bm2.reshape(1, 10)
    return _tc_tail(xp, pp, batch_p, M1x, M1p, bm1r, M2, bm2r)
